# Initial kernel scaffold; baseline (speedup 1.0000x reference)
import jax
import jax.numpy as jnp
from jax.experimental import pallas as pl


def kernel(token_ids, w):
    raise NotImplementedError("write your pallas kernel here")


# speedup vs baseline: 1.0951x; 1.0951x over previous
import functools

import jax
import jax.numpy as jnp
from jax import lax
from jax.experimental import pallas as pl
from jax.experimental.pallas import tpu as pltpu
from jax.experimental.pallas import tpu_sc as plsc

NUM_WORKERS = 32
CHUNK = 1024


def kernel(token_ids, w):
    B, S = token_ids.shape
    V, D = w.shape
    R = B * S
    idx = token_ids.reshape(R).astype(jnp.int32)

    per_w = R // NUM_WORKERS
    n_chunks = per_w // CHUNK

    mesh = plsc.VectorSubcoreMesh(core_axis_name="c", subcore_axis_name="s")

    @functools.partial(
        pl.kernel,
        mesh=mesh,
        out_type=jax.ShapeDtypeStruct((R, D), jnp.float32),
        scratch_types=[
            pltpu.VMEM((CHUNK,), jnp.int32),
            pltpu.VMEM((CHUNK, D), jnp.float32),
            pltpu.SemaphoreType.DMA,
        ],
        compiler_params=pltpu.CompilerParams(use_tc_tiling_on_sc=False),
    )
    def emb(idx_hbm, w_hbm, out_hbm, idx_v, rows_v, sem):
        wid = lax.axis_index("s") * 2 + lax.axis_index("c")
        base = wid * per_w

        def chunk_body(i, carry):
            off = base + i * CHUNK
            pltpu.sync_copy(idx_hbm.at[pl.ds(off, CHUNK)], idx_v)
            pltpu.async_copy(w_hbm.at[idx_v], rows_v, sem).wait()
            pltpu.sync_copy(rows_v, out_hbm.at[pl.ds(off, CHUNK)])
            return carry

        lax.fori_loop(0, n_chunks, chunk_body, 0)

    out = emb(idx, w)
    return out.reshape(B, S, D)


# speedup vs baseline: 1.1140x; 1.0173x over previous
import functools

import jax
import jax.numpy as jnp
from jax import lax
from jax.experimental import pallas as pl
from jax.experimental.pallas import tpu as pltpu
from jax.experimental.pallas import tpu_sc as plsc

NUM_WORKERS = 32
CHUNK = 800
NBUF = 4
NGATHER = 2


def kernel(token_ids, w):
    B, S = token_ids.shape
    V, D = w.shape
    R = B * S
    idx = token_ids.reshape(R).astype(jnp.int32)

    per_w = R // NUM_WORKERS
    n = per_w // CHUNK
    assert per_w % CHUNK == 0 and n % NBUF == 0 and n // NBUF >= 3

    mesh = plsc.VectorSubcoreMesh(core_axis_name="c", subcore_axis_name="s")

    @functools.partial(
        pl.kernel,
        mesh=mesh,
        out_type=jax.ShapeDtypeStruct((R, D), jnp.float32),
        scratch_types=[
            pltpu.VMEM((NBUF, CHUNK), jnp.int32),
            pltpu.VMEM((NBUF, CHUNK, D), jnp.float32),
            pltpu.SemaphoreType.DMA((NBUF,)),
            pltpu.SemaphoreType.DMA((NBUF,)),
            pltpu.SemaphoreType.DMA((NBUF,)),
        ],
        compiler_params=pltpu.CompilerParams(use_tc_tiling_on_sc=False),
    )
    def emb(idx_hbm, w_hbm, out_hbm, idx_v, rows_v, isem, gsem, osem):
        wid = lax.axis_index("s") * 2 + lax.axis_index("c")
        base = wid * per_w

        def start_idx(i, s):
            off = base + i * CHUNK
            pltpu.async_copy(idx_hbm.at[pl.ds(off, CHUNK)], idx_v.at[s],
                             isem.at[s])

        def wait_idx(s):
            pltpu.make_async_copy(idx_hbm.at[pl.ds(base, CHUNK)],
                                  idx_v.at[s], isem.at[s]).wait()

        def start_gather(s):
            pltpu.async_copy(w_hbm.at[idx_v.at[s]], rows_v.at[s], gsem.at[s])

        def wait_gather(s):
            pltpu.make_async_copy(w_hbm.at[idx_v.at[s]], rows_v.at[s],
                                  gsem.at[s]).wait()

        def start_out(i, s):
            off = base + i * CHUNK
            pltpu.async_copy(rows_v.at[s], out_hbm.at[pl.ds(off, CHUNK)],
                             osem.at[s])

        def wait_out(s):
            pltpu.make_async_copy(rows_v.at[s],
                                  out_hbm.at[pl.ds(base, CHUNK)],
                                  osem.at[s]).wait()

        for b in range(NBUF):
            start_idx(b, b)
        for g in range(NGATHER):
            wait_idx(g)
            start_gather(g)

        def step(i, b, first, last):
            s = b
            sg = (b + NGATHER) % NBUF
            wait_gather(s)
            start_out(i, s)
            if not last:
                start_idx(i + NBUF, s)
            if not (last and b >= NBUF - NGATHER):
                wait_idx(sg)
                if not (first and b < NBUF - NGATHER):
                    wait_out(sg)
                start_gather(sg)

        for b in range(NBUF):
            step(b, b, first=True, last=False)

        def outer(o, carry):
            for b in range(NBUF):
                step(o * NBUF + b, b, first=False, last=False)
            return carry

        lax.fori_loop(1, n // NBUF - 1, outer, 0)

        for b in range(NBUF):
            step(n - NBUF + b, b, first=False, last=True)

        for b in range(NBUF):
            wait_out(b)

    out = emb(idx, w)
    return out.reshape(B, S, D)


# speedup vs baseline: 1.5644x; 1.4043x over previous
import functools

import jax
import jax.numpy as jnp
from jax import lax
from jax.experimental import pallas as pl
from jax.experimental.pallas import tpu as pltpu
from jax.experimental.pallas import tpu_sc as plsc

NW = 32
NB1 = 4
S, B, D = 50, 16384, 32
V = 1000000
NT_FULL = 768
NROW = 800
NFB = 1536
NPB = 128


def _wid():
    return lax.axis_index("s") * 2 + lax.axis_index("c")


def _make_k1():
    mesh = plsc.VectorSubcoreMesh(core_axis_name="c", subcore_axis_name="s")

    @functools.partial(
        pl.kernel, mesh=mesh,
        out_type=jax.ShapeDtypeStruct((NROW, 8, 128), jnp.int32),
        scratch_types=[
            pltpu.VMEM((NB1, 8, 128), jnp.int32),
            pltpu.SemaphoreType.DMA((NB1,)),
            pltpu.SemaphoreType.DMA((NB1,)),
        ],
        compiler_params=pltpu.CompilerParams(use_tc_tiling_on_sc=True),
    )
    def k1(idx_hbm, idx2_hbm, vb, isem, osem):
        wid = _wid()

        def rd_src(j):
            g = j * NW + wid
            if j < 24:
                st, bt = g // 128, g % 128
                return idx_hbm.at[pl.ds(st * 8, 8), pl.ds(bt * 128, 128)]
            bt = g - NT_FULL
            return idx_hbm.at[pl.ds(48, 2), pl.ds(bt * 128, 128)]

        def wr_dst(j):
            g = j * NW + wid
            if j < 24:
                return idx2_hbm.at[g]
            bt = g - NT_FULL
            return idx2_hbm.at[NT_FULL + bt // 4, pl.ds((bt % 4) * 2, 2), :]

        def vb_ref(j):
            s = j % NB1
            return vb.at[s] if j < 24 else vb.at[s, pl.ds(0, 2), :]

        for j in range(NB1):
            pltpu.async_copy(rd_src(j), vb_ref(j), isem.at[j % NB1])
        for j in range(28):
            s = j % NB1
            pltpu.make_async_copy(rd_src(j), vb_ref(j), isem.at[s]).wait()
            pltpu.async_copy(vb_ref(j), wr_dst(j), osem.at[s])
            if j + NB1 < 28:
                pltpu.make_async_copy(vb_ref(j), wr_dst(j), osem.at[s]).wait()
                pltpu.async_copy(rd_src(j + NB1), vb_ref(j + NB1), isem.at[s])
        for j in range(24, 28):
            s = j % NB1
            pltpu.make_async_copy(vb_ref(j), wr_dst(j), osem.at[s]).wait()

    return k1


def _make_k2():
    mesh = plsc.VectorSubcoreMesh(core_axis_name="c", subcore_axis_name="s")

    @functools.partial(
        pl.kernel, mesh=mesh,
        out_type=jax.ShapeDtypeStruct((S, 4, 128, 8, 128), jnp.float32),
        scratch_types=[
            pltpu.VMEM((2, 4, 128), jnp.int32),
            pltpu.VMEM((2, 512, D), jnp.float32),
            pltpu.VMEM((2, 128, 128), jnp.float32),
            pltpu.SemaphoreType.DMA((2,)),
            pltpu.SemaphoreType.DMA((2,)),
            pltpu.SemaphoreType.DMA((2,)),
        ],
        compiler_params=pltpu.CompilerParams(use_tc_tiling_on_sc=False,
                                             needs_layout_passes=False),
    )
    def k2(idx2_hbm, w_hbm, out_hbm, ib, g, tb, isem, gsem, osem):
        wid = _wid()
        lane = lax.iota(jnp.int32, 16)

        def ib_src(k, nsub):
            hb = k * NW + wid
            if nsub == 4:
                hs, bt = hb // 128, hb % 128
                t = (hs // 2) * 128 + bt
                return idx2_hbm.at[t, pl.ds((hs % 2) * 4, 4), :]
            bt = hb
            return idx2_hbm.at[NT_FULL + bt // 4, pl.ds((bt % 4) * 2, 2), :]

        def ib_dst(sl, nsub):
            return ib.at[sl] if nsub == 4 else ib.at[sl, pl.ds(0, 2), :]

        def start_ib(k, sl, nsub):
            pltpu.async_copy(ib_src(k, nsub), ib_dst(sl, nsub), isem.at[sl])

        def wait_ib(k, sl, nsub):
            pltpu.make_async_copy(ib_src(k, nsub), ib_dst(sl, nsub),
                                  isem.at[sl]).wait()

        def start_gathers(sl, nsub):
            for su in range(nsub):
                pltpu.async_copy(w_hbm.at[ib.at[sl, su]],
                                 g.at[sl, pl.ds(su * 128, 128)], gsem.at[sl])

        def wait_gathers(sl, nsub):
            for su in range(nsub):
                pltpu.make_async_copy(w_hbm.at[ib.at[sl, su]],
                                      g.at[sl, pl.ds(su * 128, 128)],
                                      gsem.at[sl]).wait()

        def transpose(sl, nsub):
            gsl, tsl = g.at[sl], tb.at[sl]

            def body(c, carry):
                col = jnp.full((16,), c, jnp.int32)
                for su in range(nsub):
                    row = jnp.full((16,), su * 32 + c, jnp.int32)
                    for l in range(8):
                        v = plsc.load_gather(
                            gsl, [su * 128 + l * 16 + lane, col])
                        plsc.store_scatter(tsl, [row, l * 16 + lane], v)
                return carry

            lax.fori_loop(0, 32, body, 0)

        def start_outs(k, sl, nsub):
            hb = k * NW + wid
            if nsub == 4:
                hs, bt = hb // 128, hb % 128
                s0 = hs * 4
            else:
                s0, bt = 48, hb
            for su in range(nsub):
                for ct in range(4):
                    pltpu.async_copy(
                        tb.at[sl, pl.ds(su * 32 + ct * 8, 8), :],
                        out_hbm.at[s0 + su, ct, bt], osem.at[sl])

        def wait_outs(sl, nsub):
            for _ in range(4 * nsub):
                pltpu.make_async_copy(tb.at[sl, pl.ds(0, 8), :],
                                      out_hbm.at[0, 0, 0], osem.at[sl]).wait()

        nfull = NFB // NW

        def pair(i, first, last):
            a = 2 * i
            b = a + 1
            wait_ib(b, 1, 4)
            start_gathers(1, 4)
            wait_gathers(0, 4)
            if not last:
                start_ib(a + 2, 0, 4)
            if not first:
                wait_outs(0, 4)
            transpose(0, 4)
            start_outs(a, 0, 4)
            wait_gathers(1, 4)
            if not last:
                start_ib(b + 2, 1, 4)
            if not first:
                wait_outs(1, 4)
            transpose(1, 4)
            start_outs(b, 1, 4)
            if not last:
                wait_ib(a + 2, 0, 4)
                start_gathers(0, 4)
            return i

        start_ib(0, 0, 4)
        start_ib(1, 1, 4)
        wait_ib(0, 0, 4)
        start_gathers(0, 4)
        pair(0, True, False)
        lax.fori_loop(1, nfull // 2 - 1,
                      lambda i, c: pair(i, False, False), 0)
        pair(nfull // 2 - 1, False, True)
        wait_outs(0, 4)
        wait_outs(1, 4)

        for p in range(NPB // NW):
            pltpu.sync_copy(ib_src(p, 2), ib_dst(0, 2))
            start_gathers(0, 2)
            wait_gathers(0, 2)
            transpose(0, 2)
            start_outs(p, 0, 2)
            wait_outs(0, 2)

    return k2


_K1 = _make_k1()
_K2 = _make_k2()


def kernel(token_ids, w):
    assert token_ids.shape == (B, S) and w.shape == (V, D)
    idx_t = token_ids.T
    idx2 = _K1(idx_t)
    out5 = _K2(idx2, w)
    return out5.transpose(0, 1, 3, 2, 4).reshape(S, D, B).transpose(2, 0, 1)


# speedup vs baseline: 2.4017x; 1.5352x over previous
import functools

import jax
import jax.numpy as jnp
from jax import lax
from jax.experimental import pallas as pl
from jax.experimental.pallas import tpu as pltpu
from jax.experimental.pallas import tpu_sc as plsc

NW = 32
NB1 = 4
S, B, D = 50, 16384, 32
V = 1000000
NT_FULL = 768
NROW = 800
NFB = 1536
NPB = 128


def _wid():
    return lax.axis_index("s") * 2 + lax.axis_index("c")


def _make_k1():
    mesh = plsc.VectorSubcoreMesh(core_axis_name="c", subcore_axis_name="s")

    @functools.partial(
        pl.kernel, mesh=mesh,
        out_type=jax.ShapeDtypeStruct((NROW, 8, 128), jnp.int32),
        scratch_types=[
            pltpu.VMEM((NB1, 8, 128), jnp.int32),
            pltpu.SemaphoreType.DMA((NB1,)),
            pltpu.SemaphoreType.DMA((NB1,)),
        ],
        compiler_params=pltpu.CompilerParams(use_tc_tiling_on_sc=True),
    )
    def k1(idx_hbm, idx2_hbm, vb, isem, osem):
        wid = _wid()

        def rd_src(j):
            g = j * NW + wid
            if j < 24:
                st, bt = g // 128, g % 128
                return idx_hbm.at[pl.ds(st * 8, 8), pl.ds(bt * 128, 128)]
            bt = g - NT_FULL
            return idx_hbm.at[pl.ds(48, 2), pl.ds(bt * 128, 128)]

        def wr_dst(j):
            g = j * NW + wid
            if j < 24:
                return idx2_hbm.at[g]
            bt = g - NT_FULL
            return idx2_hbm.at[NT_FULL + bt // 4, pl.ds((bt % 4) * 2, 2), :]

        def vb_ref(j):
            s = j % NB1
            return vb.at[s] if j < 24 else vb.at[s, pl.ds(0, 2), :]

        for j in range(NB1):
            pltpu.async_copy(rd_src(j), vb_ref(j), isem.at[j % NB1])
        for j in range(28):
            s = j % NB1
            pltpu.make_async_copy(rd_src(j), vb_ref(j), isem.at[s]).wait()
            pltpu.async_copy(vb_ref(j), wr_dst(j), osem.at[s])
            if j + NB1 < 28:
                pltpu.make_async_copy(vb_ref(j), wr_dst(j), osem.at[s]).wait()
                pltpu.async_copy(rd_src(j + NB1), vb_ref(j + NB1), isem.at[s])
        for j in range(24, 28):
            s = j % NB1
            pltpu.make_async_copy(vb_ref(j), wr_dst(j), osem.at[s]).wait()

    return k1


def _make_k2():
    mesh = plsc.VectorSubcoreMesh(core_axis_name="c", subcore_axis_name="s")

    @functools.partial(
        pl.kernel, mesh=mesh,
        out_type=jax.ShapeDtypeStruct((S, 4, 128, 8, 128), jnp.float32),
        scratch_types=[
            pltpu.VMEM((2, 4, 128), jnp.int32),
            pltpu.VMEM((2, 512, D), jnp.float32),
            pltpu.VMEM((2, 128, 129), jnp.float32),
            pltpu.SemaphoreType.DMA((2,)),
            pltpu.SemaphoreType.DMA((2,)),
            pltpu.SemaphoreType.DMA((2,)),
        ],
        compiler_params=pltpu.CompilerParams(use_tc_tiling_on_sc=False,
                                             needs_layout_passes=False),
    )
    def k2(idx2_hbm, w_hbm, out_hbm, ib, g, tb, isem, gsem, osem):
        wid = _wid()
        lane = lax.iota(jnp.int32, 16)

        def ib_src(k, nsub):
            hb = k * NW + wid
            if nsub == 4:
                hs, bt = hb // 128, hb % 128
                t = (hs // 2) * 128 + bt
                return idx2_hbm.at[t, pl.ds((hs % 2) * 4, 4), :]
            bt = hb
            return idx2_hbm.at[NT_FULL + bt // 4, pl.ds((bt % 4) * 2, 2), :]

        def ib_dst(sl, nsub):
            return ib.at[sl] if nsub == 4 else ib.at[sl, pl.ds(0, 2), :]

        def start_ib(k, sl, nsub):
            pltpu.async_copy(ib_src(k, nsub), ib_dst(sl, nsub), isem.at[sl])

        def wait_ib(k, sl, nsub):
            pltpu.make_async_copy(ib_src(k, nsub), ib_dst(sl, nsub),
                                  isem.at[sl]).wait()

        def start_gathers(sl, nsub):
            for su in range(nsub):
                pltpu.async_copy(w_hbm.at[ib.at[sl, su]],
                                 g.at[sl, pl.ds(su * 128, 128)], gsem.at[sl])

        def wait_gathers(sl, nsub):
            for su in range(nsub):
                pltpu.make_async_copy(w_hbm.at[ib.at[sl, su]],
                                      g.at[sl, pl.ds(su * 128, 128)],
                                      gsem.at[sl]).wait()

        def transpose(sl, nsub):
            gsl, tsl = g.at[sl], tb.at[sl]

            def body(b, carry):
                col = jnp.full((16,), b, jnp.int32)
                for su in range(nsub):
                    for c0 in (0, 16):
                        v = gsl[su * 128 + b, pl.ds(c0, 16)]
                        plsc.store_scatter(tsl, [su * 32 + c0 + lane, col], v)
                return carry

            lax.fori_loop(0, 128, body, 0)

        def start_outs(k, sl, nsub):
            hb = k * NW + wid
            if nsub == 4:
                hs, bt = hb // 128, hb % 128
                s0 = hs * 4
            else:
                s0, bt = 48, hb
            for su in range(nsub):
                for ct in range(4):
                    pltpu.async_copy(
                        tb.at[sl, pl.ds(su * 32 + ct * 8, 8), pl.ds(0, 128)],
                        out_hbm.at[s0 + su, ct, bt], osem.at[sl])

        def wait_outs(sl, nsub):
            for _ in range(4 * nsub):
                pltpu.make_async_copy(tb.at[sl, pl.ds(0, 8), pl.ds(0, 128)],
                                      out_hbm.at[0, 0, 0], osem.at[sl]).wait()

        nfull = NFB // NW

        def pair(i, first, last):
            a = 2 * i
            b = a + 1
            wait_ib(b, 1, 4)
            start_gathers(1, 4)
            wait_gathers(0, 4)
            if not last:
                start_ib(a + 2, 0, 4)
            if not first:
                wait_outs(0, 4)
            transpose(0, 4)
            start_outs(a, 0, 4)
            wait_gathers(1, 4)
            if not last:
                start_ib(b + 2, 1, 4)
            if not first:
                wait_outs(1, 4)
            transpose(1, 4)
            start_outs(b, 1, 4)
            if not last:
                wait_ib(a + 2, 0, 4)
                start_gathers(0, 4)
            return i

        start_ib(0, 0, 4)
        start_ib(1, 1, 4)
        wait_ib(0, 0, 4)
        start_gathers(0, 4)
        pair(0, True, False)
        lax.fori_loop(1, nfull // 2 - 1,
                      lambda i, c: pair(i, False, False), 0)
        pair(nfull // 2 - 1, False, True)
        wait_outs(0, 4)
        wait_outs(1, 4)

        for p in range(NPB // NW):
            pltpu.sync_copy(ib_src(p, 2), ib_dst(0, 2))
            start_gathers(0, 2)
            wait_gathers(0, 2)
            transpose(0, 2)
            start_outs(p, 0, 2)
            wait_outs(0, 2)

    return k2


_K1 = _make_k1()
_K2 = _make_k2()


def kernel(token_ids, w):
    assert token_ids.shape == (B, S) and w.shape == (V, D)
    idx_t = token_ids.T
    idx2 = _K1(idx_t)
    out5 = _K2(idx2, w)
    return out5.transpose(0, 1, 3, 2, 4).reshape(S, D, B).transpose(2, 0, 1)
